# Initial kernel scaffold; baseline (speedup 1.0000x reference)
import jax
import jax.numpy as jnp
from jax.experimental import pallas as pl


def kernel(class_outs, regression_outs, boxes, images_shape):
    raise NotImplementedError("write your pallas kernel here")


# speedup vs baseline: 2.1590x; 2.1590x over previous
import jax
import jax.numpy as jnp
import numpy as np
from jax.experimental import pallas as pl
from jax.experimental.pallas import tpu as pltpu

_C = 80
_SCORE_THR = 0.05
_IOU_THR = 0.5
_MAX_PER_IMG = 100
_PRE_NMS = 2000
_K = 2048
_ROWS, _COLS = 8, 256
_MAX_RATIO = float(np.abs(np.log(16.0 / 1000.0)))


def _softmax_mask_kernel(x_ref, o_ref):
    x = x_ref[...]
    m = jnp.max(x, axis=-1, keepdims=True)
    e = jnp.exp(x - m)
    s = jnp.sum(e, axis=-1, keepdims=True)
    p = e[:, :_C] / s
    o_ref[...] = jnp.where(p > _SCORE_THR, p, -1.0)


def _decode(rx1, ry1, rx2, ry2, d0, d1, d2, d3, w, h):
    dx = d0 * 0.1
    dy = d1 * 0.1
    dw = jnp.clip(d2 * 0.2, -_MAX_RATIO, _MAX_RATIO)
    dh = jnp.clip(d3 * 0.2, -_MAX_RATIO, _MAX_RATIO)
    px = (rx1 + rx2) * 0.5
    py = (ry1 + ry2) * 0.5
    pw = rx2 - rx1
    ph = ry2 - ry1
    gx = px + pw * dx
    gy = py + ph * dy
    gw = pw * jnp.exp(dw)
    gh = ph * jnp.exp(dh)
    x1 = jnp.clip(gx - gw * 0.5, 0.0, w)
    y1 = jnp.clip(gy - gh * 0.5, 0.0, h)
    x2 = jnp.clip(gx + gw * 0.5, 0.0, w)
    y2 = jnp.clip(gy + gh * 0.5, 0.0, h)
    return x1, y1, x2, y2


def _nms_kernel(wh_ref,
                sc_v_ref, lab_v_ref,
                rx1_v, ry1_v, rx2_v, ry2_v, d0_v, d1_v, d2_v, d3_v,
                lab_s_ref,
                rx1_s, ry1_s, rx2_s, ry2_s, d0_s, d1_s, d2_s, d3_s,
                fs_ref, bx1_ref, by1_ref, bx2_ref, by2_ref,
                sx1_ref, sy1_ref, sx2_ref, sy2_ref):
    w = wh_ref[0, 0]
    h = wh_ref[0, 1]

    cx1, cy1, cx2, cy2 = _decode(rx1_v[...], ry1_v[...], rx2_v[...], ry2_v[...],
                                 d0_v[...], d1_v[...], d2_v[...], d3_v[...], w, h)
    bx1_ref[...] = cx1
    by1_ref[...] = cy1
    bx2_ref[...] = cx2
    by2_ref[...] = cy2

    off_base = jnp.maximum(jnp.maximum(jnp.max(cx1), jnp.max(cy1)),
                           jnp.maximum(jnp.max(cx2), jnp.max(cy2))) + 1.0
    lab = lab_v_ref[...]
    ox1 = cx1 + lab * off_base
    oy1 = cy1 + lab * off_base
    ox2 = cx2 + lab * off_base
    oy2 = cy2 + lab * off_base
    area = (cx2 - cx1) * (cy2 - cy1)

    scx1, scy1, scx2, scy2 = _decode(rx1_s[...], ry1_s[...], rx2_s[...], ry2_s[...],
                                     d0_s[...], d1_s[...], d2_s[...], d3_s[...], w, h)
    lab_s = lab_s_ref[...]
    sx1_ref[...] = scx1 + lab_s * off_base
    sy1_ref[...] = scy1 + lab_s * off_base
    sx2_ref[...] = scx2 + lab_s * off_base
    sy2_ref[...] = scy2 + lab_s * off_base

    rr = jax.lax.broadcasted_iota(jnp.int32, (_ROWS, _COLS), 0)
    cc = jax.lax.broadcasted_iota(jnp.int32, (_ROWS, _COLS), 1)
    idx = rr * _COLS + cc

    scores = sc_v_ref[...]
    keep0 = jnp.where(scores > _SCORE_THR, 1.0, 0.0)

    def body(i, keep):
        x1i = sx1_ref[pl.ds(i, 1), :]
        y1i = sy1_ref[pl.ds(i, 1), :]
        x2i = sx2_ref[pl.ds(i, 1), :]
        y2i = sy2_ref[pl.ds(i, 1), :]
        area_i = (x2i - x1i) * (y2i - y1i)
        alive = jnp.max(jnp.where(idx == i, keep, 0.0))
        xx1 = jnp.maximum(ox1, x1i)
        yy1 = jnp.maximum(oy1, y1i)
        xx2 = jnp.minimum(ox2, x2i)
        yy2 = jnp.minimum(oy2, y2i)
        inter = jnp.maximum(xx2 - xx1, 0.0) * jnp.maximum(yy2 - yy1, 0.0)
        iou = inter / (area + area_i - inter + 1e-6)
        sup = jnp.where((iou > _IOU_THR) & (idx > i), alive, 0.0)
        return keep * (1.0 - sup)

    keep = jax.lax.fori_loop(0, _PRE_NMS, body, keep0)
    fs_ref[...] = jnp.where(keep > 0.5, scores, -1.0)


def kernel(class_outs, regression_outs, boxes, images_shape):
    n, c1 = class_outs.shape
    h = images_shape[2].astype(jnp.float32)
    w = images_shape[3].astype(jnp.float32)

    block_rows = 2000
    masked = pl.pallas_call(
        _softmax_mask_kernel,
        grid=(n // block_rows,),
        in_specs=[pl.BlockSpec((block_rows, c1), lambda i: (i, 0))],
        out_specs=pl.BlockSpec((block_rows, _C), lambda i: (i, 0)),
        out_shape=jax.ShapeDtypeStruct((n, _C), jnp.float32),
    )(class_outs)

    flat = masked.reshape(-1)
    top_s, top_i = jax.lax.top_k(flat, _PRE_NMS)
    prop = top_i // _C
    lab = top_i % _C
    rois = boxes[prop]
    reg = regression_outs.reshape(-1, 4)[top_i]

    pad = _K - _PRE_NMS
    sc_p = jnp.pad(top_s, (0, pad), constant_values=-1.0)
    lab_p = jnp.pad(lab.astype(jnp.float32), (0, pad))
    rois_p = jnp.pad(rois, ((0, pad), (0, 0)))
    reg_p = jnp.pad(reg, ((0, pad), (0, 0)))

    def v(a):
        return a.reshape(_ROWS, _COLS)

    def s(a):
        return a.reshape(_K, 1)

    wh = jnp.zeros((1, 128), jnp.float32).at[0, 0].set(w).at[0, 1].set(h)

    r_cols = [rois_p[:, j] for j in range(4)]
    d_cols = [reg_p[:, j] for j in range(4)]

    out_shapes = [jax.ShapeDtypeStruct((_ROWS, _COLS), jnp.float32)] * 5
    fs, bx1, by1, bx2, by2 = pl.pallas_call(
        _nms_kernel,
        out_shape=out_shapes,
        scratch_shapes=[pltpu.VMEM((_K, 1), jnp.float32)] * 4,
    )(wh,
      v(sc_p), v(lab_p),
      v(r_cols[0]), v(r_cols[1]), v(r_cols[2]), v(r_cols[3]),
      v(d_cols[0]), v(d_cols[1]), v(d_cols[2]), v(d_cols[3]),
      s(lab_p),
      s(r_cols[0]), s(r_cols[1]), s(r_cols[2]), s(r_cols[3]),
      s(d_cols[0]), s(d_cols[1]), s(d_cols[2]), s(d_cols[3]))

    fs_flat = fs.reshape(-1)[:_PRE_NMS]
    det_scores, det_i = jax.lax.top_k(fs_flat, _MAX_PER_IMG)
    cand_boxes = jnp.stack([bx1.reshape(-1)[:_PRE_NMS],
                            by1.reshape(-1)[:_PRE_NMS],
                            bx2.reshape(-1)[:_PRE_NMS],
                            by2.reshape(-1)[:_PRE_NMS]], axis=1)
    det_boxes = cand_boxes[det_i]
    det_labels = lab[det_i].astype(jnp.int32)
    return det_boxes, det_scores, det_labels
